# Initial kernel scaffold; baseline (speedup 1.0000x reference)
import jax
import jax.numpy as jnp
from jax.experimental import pallas as pl


def kernel(seq_id, item_mask, feat_cate, feat_brand, feat_price, feat_mm, emb_item, emb_cate, emb_brand, mm_W, mm_b, W1, b1, W2, b2):
    raise NotImplementedError("write your pallas kernel here")


# speedup vs baseline: 3.9947x; 3.9947x over previous
import functools
import jax
import jax.numpy as jnp
from jax import lax
from jax.experimental import pallas as pl
from jax.experimental.pallas import tpu as pltpu
from jax.experimental.pallas import tpu_sc as plsc

_NC = 2
_NS = 16
_NW = _NC * _NS
_CHUNK = 128


def _gather_body(nch, d_e, idx_hbm, t_item, t_cate, t_brand,
                 out_item, out_cate, out_brand,
                 idx_v, rows_a, rows_b, rows_c, sem):
  wid = lax.axis_index("s") * _NC + lax.axis_index("c")
  base = wid * nch * _CHUNK
  pltpu.sync_copy(idx_hbm.at[0, wid], idx_v.at[0])
  pltpu.sync_copy(idx_hbm.at[1, wid], idx_v.at[1])
  pltpu.sync_copy(idx_hbm.at[2, wid], idx_v.at[2])

  def chunk_body(c, _):
    row0 = base + c * _CHUNK
    cp_a = pltpu.async_copy(t_item.at[idx_v.at[0, c]], rows_a, sem)
    cp_b = pltpu.async_copy(t_cate.at[idx_v.at[1, c]], rows_b, sem)
    cp_c = pltpu.async_copy(t_brand.at[idx_v.at[2, c]], rows_c, sem)
    cp_a.wait()
    pltpu.sync_copy(rows_a, out_item.at[pl.ds(row0, _CHUNK)])
    cp_b.wait()
    pltpu.sync_copy(rows_b, out_cate.at[pl.ds(row0, _CHUNK)])
    cp_c.wait()
    pltpu.sync_copy(rows_c, out_brand.at[pl.ds(row0, _CHUNK)])
    return 0

  lax.fori_loop(0, nch, chunk_body, 0)


def _sc_gather(idx, emb_item, emb_cate, emb_brand):
  _, nw, nch, chunk = idx.shape
  n = nw * nch * chunk
  d_e = emb_item.shape[1]
  mesh = plsc.VectorSubcoreMesh(core_axis_name="c", subcore_axis_name="s",
                                num_cores=_NC, num_subcores=_NS)
  out_sds = jax.ShapeDtypeStruct((n, d_e), jnp.float32)
  body = functools.partial(_gather_body, nch, d_e)
  return pl.kernel(
      body,
      out_type=[out_sds, out_sds, out_sds],
      mesh=mesh,
      scratch_types=[
          pltpu.VMEM((3, nch, chunk), jnp.int32),
          pltpu.VMEM((chunk, d_e), jnp.float32),
          pltpu.VMEM((chunk, d_e), jnp.float32),
          pltpu.VMEM((chunk, d_e), jnp.float32),
          pltpu.SemaphoreType.DMA,
      ],
      compiler_params=pltpu.CompilerParams(use_tc_tiling_on_sc=False),
  )(idx, emb_item, emb_cate, emb_brand)


def _mlp_body(ei, ec, eb, pr, mm, wc, b1e, w2, b2, out):
  feats = jnp.concatenate(
      [ei[...], ec[...], eb[...], pr[...], mm[...]], axis=1)
  h = jnp.dot(feats, wc[...], preferred_element_type=jnp.float32) + b1e[...]
  h = jnp.maximum(h, 0.0)
  out[...] = jnp.dot(h, w2[...], preferred_element_type=jnp.float32) + b2[...]


def _tc_mlp(ei, ec, eb, pr, mm, wcat, b1e, w2, b2, bl=512):
  n, d_e = ei.shape
  d_mm = mm.shape[1]
  d_cat, d_dnn = wcat.shape
  d_hid = w2.shape[1]
  grid = (n // bl,)
  full = lambda r, c: pl.BlockSpec((r, c), lambda i: (0, 0))
  return pl.pallas_call(
      _mlp_body,
      grid=grid,
      in_specs=[
          pl.BlockSpec((bl, d_e), lambda i: (i, 0)),
          pl.BlockSpec((bl, d_e), lambda i: (i, 0)),
          pl.BlockSpec((bl, d_e), lambda i: (i, 0)),
          pl.BlockSpec((bl, 1), lambda i: (i, 0)),
          pl.BlockSpec((bl, d_mm), lambda i: (i, 0)),
          full(d_cat, d_dnn),
          full(1, d_dnn),
          full(d_dnn, d_hid),
          full(1, d_hid),
      ],
      out_specs=pl.BlockSpec((bl, d_hid), lambda i: (i, 0)),
      out_shape=jax.ShapeDtypeStruct((n, d_hid), jnp.float32),
  )(ei, ec, eb, pr, mm, wcat, b1e, w2, b2)


def kernel(seq_id, item_mask, feat_cate, feat_brand, feat_price, feat_mm,
           emb_item, emb_cate, emb_brand, mm_W, mm_b, W1, b1, W2, b2):
  b, l = seq_id.shape
  n = b * l
  d_e = emb_item.shape[1]
  d_mm_in, d_mm = mm_W.shape
  d_hid = W2.shape[1]
  assert n % (_NW * _CHUNK) == 0
  nch = n // (_NW * _CHUNK)

  ids = (seq_id * item_mask).astype(jnp.int32).reshape(-1)
  idx = jnp.stack([
      ids,
      feat_cate.astype(jnp.int32).reshape(-1),
      feat_brand.astype(jnp.int32).reshape(-1),
  ]).reshape(3, _NW, nch, _CHUNK)

  ei, ec, ebr = _sc_gather(idx, emb_item, emb_cate, emb_brand)

  w1_mm = W1[d_e * 3 + 1:]
  wcat = jnp.concatenate([W1[:d_e * 3 + 1], mm_W @ w1_mm], axis=0)
  b1e = (b1 + mm_b @ w1_mm)[None, :]

  out = _tc_mlp(ei, ec, ebr,
                feat_price.reshape(n, 1).astype(jnp.float32),
                feat_mm.reshape(n, d_mm_in),
                wcat, b1e, W2, b2[None, :])
  return out.reshape(b, l, d_hid)


# speedup vs baseline: 5.1635x; 1.2926x over previous
import functools
import jax
import jax.numpy as jnp
from jax import lax
from jax.experimental import pallas as pl
from jax.experimental.pallas import tpu as pltpu
from jax.experimental.pallas import tpu_sc as plsc

_NC = 2
_NS = 16
_NW = _NC * _NS
_CHUNK = 128


_GG = 5


def _gather_body(nch, d_e, idx_i_hbm, idx_c_hbm, idx_b_hbm,
                 t_item, t_cate, t_brand,
                 out_item, out_cate, out_brand,
                 idx_v, rows_i, rows_c, rows_b, gsem, wsem):
  wid = lax.axis_index("s") * _NC + lax.axis_index("c")
  base = wid * nch * _CHUNK
  pltpu.sync_copy(idx_i_hbm.at[wid], idx_v.at[0])
  pltpu.sync_copy(idx_c_hbm.at[wid], idx_v.at[1])
  pltpu.sync_copy(idx_b_hbm.at[wid], idx_v.at[2])

  tables = ((t_item, rows_i, out_item, 0),
            (t_cate, rows_c, out_cate, 1),
            (t_brand, rows_b, out_brand, 2))

  def group_body(g, _):
    row0 = base + g * (_GG * _CHUNK)
    cps = []
    for tbl, rows, _out, t in tables:
      for j in range(_GG):
        cps.append(pltpu.async_copy(
            tbl.at[idx_v.at[t, g * _GG + j]],
            rows.at[pl.ds(j * _CHUNK, _CHUNK)], gsem))
    for cp in cps:
      cp.wait()
    wr = [pltpu.async_copy(rows, _out.at[pl.ds(row0, _GG * _CHUNK)], wsem)
          for _tbl, rows, _out, _t in tables]
    for cp in wr:
      cp.wait()
    return 0

  lax.fori_loop(0, nch // _GG, group_body, 0)


def _sc_gather(idx_i, idx_c, idx_b, emb_item, emb_cate, emb_brand):
  nw, nch, chunk = idx_i.shape
  n = nw * nch * chunk
  d_e = emb_item.shape[1]
  mesh = plsc.VectorSubcoreMesh(core_axis_name="c", subcore_axis_name="s",
                                num_cores=_NC, num_subcores=_NS)
  out_sds = jax.ShapeDtypeStruct((n, d_e), jnp.float32)
  rows_t = pltpu.VMEM((_GG * chunk, d_e), jnp.float32)
  body = functools.partial(_gather_body, nch, d_e)
  return pl.kernel(
      body,
      out_type=[out_sds, out_sds, out_sds],
      mesh=mesh,
      scratch_types=[
          pltpu.VMEM((3, nch, chunk), jnp.int32),
          rows_t, rows_t, rows_t,
          pltpu.SemaphoreType.DMA,
          pltpu.SemaphoreType.DMA,
      ],
      compiler_params=pltpu.CompilerParams(use_tc_tiling_on_sc=False),
  )(idx_i, idx_c, idx_b, emb_item, emb_cate, emb_brand)


def _mlp_body(ei, ec, eb, pr, mm, wc, b1e, w2, b2, out):
  feats = jnp.concatenate(
      [ei[...], ec[...], eb[...], pr[...], mm[...]], axis=1)
  h = jnp.dot(feats, wc[...], preferred_element_type=jnp.float32) + b1e[...]
  h = jnp.maximum(h, 0.0)
  out[...] = jnp.dot(h, w2[...], preferred_element_type=jnp.float32) + b2[...]


def _tc_mlp(ei, ec, eb, pr, mm, wcat, b1e, w2, b2, bl=512):
  n, d_e = ei.shape
  d_mm = mm.shape[1]
  d_cat, d_dnn = wcat.shape
  d_hid = w2.shape[1]
  grid = (n // bl,)
  full = lambda r, c: pl.BlockSpec((r, c), lambda i: (0, 0))
  return pl.pallas_call(
      _mlp_body,
      grid=grid,
      in_specs=[
          pl.BlockSpec((bl, d_e), lambda i: (i, 0)),
          pl.BlockSpec((bl, d_e), lambda i: (i, 0)),
          pl.BlockSpec((bl, d_e), lambda i: (i, 0)),
          pl.BlockSpec((bl, 1), lambda i: (i, 0)),
          pl.BlockSpec((bl, d_mm), lambda i: (i, 0)),
          full(d_cat, d_dnn),
          full(1, d_dnn),
          full(d_dnn, d_hid),
          full(1, d_hid),
      ],
      out_specs=pl.BlockSpec((bl, d_hid), lambda i: (i, 0)),
      out_shape=jax.ShapeDtypeStruct((n, d_hid), jnp.float32),
  )(ei, ec, eb, pr, mm, wcat, b1e, w2, b2)


def kernel(seq_id, item_mask, feat_cate, feat_brand, feat_price, feat_mm,
           emb_item, emb_cate, emb_brand, mm_W, mm_b, W1, b1, W2, b2):
  b, l = seq_id.shape
  n = b * l
  d_e = emb_item.shape[1]
  d_mm_in, d_mm = mm_W.shape
  d_hid = W2.shape[1]
  assert n % (_NW * _CHUNK) == 0
  nch = n // (_NW * _CHUNK)

  ids = (seq_id * item_mask).astype(jnp.int32).reshape(_NW, nch, _CHUNK)
  idx_c = feat_cate.astype(jnp.int32).reshape(_NW, nch, _CHUNK)
  idx_b = feat_brand.astype(jnp.int32).reshape(_NW, nch, _CHUNK)

  ei, ec, ebr = _sc_gather(ids, idx_c, idx_b, emb_item, emb_cate, emb_brand)

  w1_mm = W1[d_e * 3 + 1:]
  wcat = jnp.concatenate([W1[:d_e * 3 + 1], mm_W @ w1_mm], axis=0)
  b1e = (b1 + mm_b @ w1_mm)[None, :]

  out = _tc_mlp(ei, ec, ebr,
                feat_price.reshape(n, 1).astype(jnp.float32),
                feat_mm.reshape(n, d_mm_in),
                wcat, b1e, W2, b2[None, :])
  return out.reshape(b, l, d_hid)
